# baseline (speedup 1.0000x reference)
import functools

import jax
import jax.numpy as jnp
from jax import lax
from jax.experimental import pallas as pl
from jax.experimental.pallas import tpu as pltpu
from jax.experimental.pallas import tpu_sc as plsc

_C = 100000
_B = 1024
_IGNORE = 0
_SMOOTHING = 0.1
_CONF = 1.0 - _SMOOTHING
_EPS = _SMOOTHING / (_C - 1)

_NC = 2
_NS = 16
_L = 16
_NW = _NC * _NS
_BPW = _B // _NW

_BR = 16
_NB = _B // _BR


def _sc_gather_body(logit_flat, tgt, out, tgt_v, idx_v, val_v, sem):
    wid = lax.axis_index("s") * _NC + lax.axis_index("c")
    base = wid * _BPW
    pltpu.sync_copy(tgt.at[pl.ds(base, _BPW)], tgt_v)
    for i in range(_BPW // _L):
        t = tgt_v[pl.ds(i * _L, _L)]
        rows = (base + i * _L) + lax.iota(jnp.int32, _L)
        idx_v[pl.ds(i * _L, _L)] = rows + 0 * t
    pltpu.async_copy(logit_flat.at[idx_v], val_v, sem).wait()
    for i in range(_BPW // _L):
        t = tgt_v[pl.ds(i * _L, _L)]
        v = val_v[pl.ds(i * _L, _L)]
        val_v[pl.ds(i * _L, _L)] = jnp.where(t != _IGNORE, v, 0.0)
    pltpu.sync_copy(val_v, out.at[pl.ds(base, _BPW)])


@functools.lru_cache(maxsize=1)
def _sc_gather():
    return pl.kernel(
        _sc_gather_body,
        mesh=plsc.VectorSubcoreMesh(core_axis_name="c", subcore_axis_name="s"),
        out_type=jax.ShapeDtypeStruct((_B,), jnp.float32),
        scratch_types=[
            pltpu.VMEM((_BPW,), jnp.int32),
            pltpu.VMEM((_BPW,), jnp.int32),
            pltpu.VMEM((_BPW,), jnp.float32),
            pltpu.SemaphoreType.DMA,
        ],
    )


_NBUF = 4


def _tc_reduce_body(tgt_ref, y_ref, hbm_ref, o_ref, bufs, sems):
    j = pl.program_id(0)
    slot = lax.rem(j, _NBUF)

    @pl.when(j == 0)
    def _():
        o_ref[0, 0] = (_CONF - _EPS) * jnp.sum(y_ref[...])
        for k in range(_NBUF):
            pltpu.make_async_copy(
                hbm_ref.at[pl.ds(k * _BR, _BR), :], bufs.at[k], sems.at[k]
            ).start()

    pltpu.make_async_copy(
        hbm_ref.at[pl.ds(j * _BR, _BR), :], bufs.at[slot], sems.at[slot]
    ).wait()
    w = (tgt_ref[...] != _IGNORE).astype(jnp.float32)
    o_ref[0, 0] += _EPS * jnp.sum(bufs[slot] * w)

    nxt = j + _NBUF

    @pl.when(nxt < _NB)
    def _():
        pltpu.make_async_copy(
            hbm_ref.at[pl.ds(nxt * _BR, _BR), :], bufs.at[slot], sems.at[slot]
        ).start()


def kernel(logit, target):
    y = _sc_gather()(logit[:, 0].reshape(-1), target)
    out = pl.pallas_call(
        _tc_reduce_body,
        grid=(_NB,),
        in_specs=[
            pl.BlockSpec((_BR, 1), lambda j: (j, 0)),
            pl.BlockSpec((8, 128), lambda j: (0, 0)),
            pl.BlockSpec(memory_space=pltpu.HBM),
        ],
        out_specs=pl.BlockSpec(memory_space=pltpu.SMEM),
        out_shape=jax.ShapeDtypeStruct((1, 1), jnp.float32),
        scratch_shapes=[
            pltpu.VMEM((_NBUF, _BR, _C), jnp.float32),
            pltpu.SemaphoreType.DMA((_NBUF,)),
        ],
    )(target.reshape(_B, 1), y.reshape(8, 128), logit)
    return -out[0, 0]


# speedup vs baseline: 1.0346x; 1.0346x over previous
import functools

import jax
import jax.numpy as jnp
from jax import lax
from jax.experimental import pallas as pl
from jax.experimental.pallas import tpu as pltpu
from jax.experimental.pallas import tpu_sc as plsc

_C = 100000
_B = 1024
_IGNORE = 0
_SMOOTHING = 0.1
_CONF = 1.0 - _SMOOTHING
_EPS = _SMOOTHING / (_C - 1)

_NC = 2
_NS = 16
_L = 16
_NW = _NC * _NS
_BPW = _B // _NW

_BR = 16
_NB = _B // _BR


def _sc_gather_body(logit_hbm, tgt, out, tgt_v, tiles_v, val_v, sem):
    wid = lax.axis_index("s") * _NC + lax.axis_index("c")
    base = wid * _BPW
    pltpu.sync_copy(tgt.at[pl.ds(base, _BPW)], tgt_v)
    lanes = lax.iota(jnp.int32, _L)
    tvecs = [tgt_v[pl.ds(k * _L, _L)] for k in range(_BPW // _L)]
    tscal = []
    for i in range(_BPW):
        t = jnp.sum(jnp.where(lanes == (i % _L), tvecs[i // _L], 0))
        tscal.append(t)
        row8 = base + (i // 8) * 8
        col128 = (t // 128) * 128
        pltpu.async_copy(
            logit_hbm.at[pl.ds(row8, 8), pl.ds(col128, 128)],
            tiles_v.at[i],
            sem,
        ).start()
    for i in range(_BPW):
        pltpu.make_async_copy(
            logit_hbm.at[pl.ds(0, 8), pl.ds(0, 128)], tiles_v.at[i], sem
        ).wait()
    for k in range(_BPW // _L):
        acc = jnp.zeros((_L,), jnp.float32)
        for j in range(_L):
            i = k * _L + j
            t = tscal[i]
            sub = (base + i) % 8
            l16 = ((t % 128) // 16) * 16
            vec = tiles_v[i, sub, pl.ds(l16, 16)]
            y = jnp.sum(jnp.where(lanes == (t % 16), vec, 0.0))
            y = jnp.where(t != _IGNORE, y, 0.0)
            acc = jnp.where(lanes == j, y, acc)
        val_v[pl.ds(k * _L, _L)] = acc
    pltpu.sync_copy(val_v, out.at[pl.ds(base, _BPW)])


@functools.lru_cache(maxsize=1)
def _sc_gather():
    return pl.kernel(
        _sc_gather_body,
        mesh=plsc.VectorSubcoreMesh(core_axis_name="c", subcore_axis_name="s"),
        compiler_params=pltpu.CompilerParams(needs_layout_passes=False),
        out_type=jax.ShapeDtypeStruct((_B,), jnp.float32),
        scratch_types=[
            pltpu.VMEM((_BPW,), jnp.int32),
            pltpu.VMEM((_BPW, 8, 128), jnp.float32),
            pltpu.VMEM((_BPW,), jnp.float32),
            pltpu.SemaphoreType.DMA,
        ],
    )


def _tc_reduce_body(tgt1_ref, tgt2_ref, y_ref, x1_ref, x2_ref, o_ref):
    j = pl.program_id(0)

    @pl.when(j == 0)
    def _():
        o_ref[0, 0] = (_CONF - _EPS) * jnp.sum(y_ref[...])

    w1 = (tgt1_ref[...] != _IGNORE).astype(jnp.float32)
    w2 = (tgt2_ref[...] != _IGNORE).astype(jnp.float32)
    o_ref[0, 0] += _EPS * (jnp.sum(x1_ref[...] * w1) + jnp.sum(x2_ref[...] * w2))


def kernel(logit, target):
    y = _sc_gather()(logit, target)
    tgt2d = target.reshape(_B, 1)
    half = _NB // 2
    out = pl.pallas_call(
        _tc_reduce_body,
        grid=(half,),
        in_specs=[
            pl.BlockSpec((_BR, 1), lambda j: (j, 0)),
            pl.BlockSpec((_BR, 1), lambda j: (j + half, 0)),
            pl.BlockSpec((8, 128), lambda j: (0, 0)),
            pl.BlockSpec((_BR, _C), lambda j: (j, 0)),
            pl.BlockSpec((_BR, _C), lambda j: (j + half, 0)),
        ],
        out_specs=pl.BlockSpec(memory_space=pltpu.SMEM),
        out_shape=jax.ShapeDtypeStruct((1, 1), jnp.float32),
    )(tgt2d, tgt2d, y.reshape(8, 128), logit, logit)
    return -out[0, 0]
